# baseline (speedup 1.0000x reference)
import functools

import jax
import jax.numpy as jnp
from jax import lax
from jax.experimental import pallas as pl
from jax.experimental.pallas import tpu as pltpu
from jax.experimental.pallas import tpu_sc as plsc

N = 10000
E = 160000
H = 256
B = 1024
R = 64
V = 50000
T = 4

NC = 2
NS = 16
NW = NC * NS

NP = 10240
TOK_PAD = NP * T
E_PAD = 163840
ROWS_SC = NP // NS
HH = H // 2
HQ = H // 4
BLK = 1024

F32 = jnp.float32


def _dot_nt(a, b):
    return lax.dot_general(a, b, (((1,), (1,)), ((), ())),
                           preferred_element_type=F32)


def _dot_nn(a, b):
    return lax.dot_general(a, b, (((1,), (0,)), ((), ())),
                           preferred_element_type=F32)


@functools.lru_cache(maxsize=None)
def _mesh():
    return plsc.VectorSubcoreMesh(core_axis_name="c", subcore_axis_name="s",
                                  num_cores=NC, num_subcores=NS)


def _k1_body(tok_idx, wemb, out, idx_v, rows_v, sem):
    wid = lax.axis_index("s") * NC + lax.axis_index("c")
    per = TOK_PAD // NW
    base = wid * per

    def chunk(j, carry):
        off = base + j * 128
        pltpu.sync_copy(tok_idx.at[pl.ds(off, 128)], idx_v)
        pltpu.async_copy(wemb.at[idx_v], rows_v, sem).wait()
        pltpu.sync_copy(rows_v, out.at[pl.ds(off, 128)])
        return carry

    lax.fori_loop(0, per // 128, chunk, 0)


@functools.lru_cache(maxsize=None)
def _k1_kernel():
    return pl.kernel(
        _k1_body,
        out_type=jax.ShapeDtypeStruct((TOK_PAD, H), F32),
        mesh=_mesh(),
        scratch_types=[
            pltpu.VMEM((128,), jnp.int32),
            pltpu.VMEM((128, H), F32),
            pltpu.SemaphoreType.DMA,
        ],
    )


def _k1_call(tok_idx, wemb):
    return _k1_kernel()(tok_idx, wemb)


def _k2_body(tok2, wmsg, wh, bh, dst2, nf_o, x1a_o, x1b_o, gh_o, dlo_o, dhi_o):
    t = tok2[...]
    nf = (t[:, 0:H] + t[:, H:2 * H] + t[:, 2 * H:3 * H] + t[:, 3 * H:4 * H]) * 0.25
    x1 = _dot_nn(nf, wmsg[...])
    x1a_o[...] = x1[:, :HH]
    x1b_o[...] = x1[:, HH:]
    gh_o[...] = _dot_nt(nf, wh[...]) + bh[...]
    nf_o[...] = nf
    d = dst2[...]
    dlo_o[...] = jnp.where(d < NHALF, d, NHALF)
    dhi_o[...] = jnp.where(d >= NHALF, d - NHALF, NHALF)


def _k2_call(tok2, W_msg, Wh, bh_row, dst2):
    grid = (NP // BLK,)
    return pl.pallas_call(
        _k2_body,
        grid=grid,
        in_specs=[
            pl.BlockSpec((BLK, 4 * H), lambda i: (i, 0)),
            pl.BlockSpec((H, H), lambda i: (0, 0)),
            pl.BlockSpec((3 * H, H), lambda i: (0, 0)),
            pl.BlockSpec((1, 3 * H), lambda i: (0, 0)),
            pl.BlockSpec((E_PAD // 128 // (NP // BLK), 128), lambda i: (i, 0)),
        ],
        out_specs=[
            pl.BlockSpec((BLK, H), lambda i: (i, 0)),
            pl.BlockSpec((BLK, HH), lambda i: (i, 0)),
            pl.BlockSpec((BLK, HH), lambda i: (i, 0)),
            pl.BlockSpec((BLK, 3 * H), lambda i: (i, 0)),
            pl.BlockSpec((E_PAD // 128 // (NP // BLK), 128), lambda i: (i, 0)),
            pl.BlockSpec((E_PAD // 128 // (NP // BLK), 128), lambda i: (i, 0)),
        ],
        out_shape=[
            jax.ShapeDtypeStruct((NP, H), F32),
            jax.ShapeDtypeStruct((NP, HH), F32),
            jax.ShapeDtypeStruct((NP, HH), F32),
            jax.ShapeDtypeStruct((NP, 3 * H), F32),
            jax.ShapeDtypeStruct((E_PAD // 128, 128), jnp.int32),
            jax.ShapeDtypeStruct((E_PAD // 128, 128), jnp.int32),
        ],
    )(tok2, W_msg, Wh, bh_row, dst2)


NBUF = 4
CH_W = (E_PAD // NS) // 128


NHALF = NP // 2
SLAB = NHALF + 128
ROWS_Z = SLAB // NS
ROWS_W = NHALF // NS


def _k3_body(src2, dlo2, dhi2, table, zrows, out_ref,
             idx_s, idx_d, rows0, rows1, rows2, rows3,
             sem0, sem1, sem2, sem3, shared):
    sid = lax.axis_index("s")
    rows = (rows0, rows1, rows2, rows3)
    sems = (sem0, sem1, sem2, sem3)

    row_base = sid * CH_W
    pltpu.sync_copy(src2.at[pl.ds(row_base, CH_W)], idx_s)

    def phase(dst2, out_base):
        pltpu.sync_copy(dst2.at[pl.ds(row_base, CH_W)], idx_d)
        pltpu.sync_copy(zrows, shared.at[pl.ds(sid * ROWS_Z, ROWS_Z)])
        plsc.subcore_barrier()

        def group(g, carry):
            j0 = g * NBUF
            descs = []
            for b in range(NBUF):
                descs.append(pltpu.async_copy(
                    table.at[idx_s.at[j0 + b]], rows[b], sems[b]))
            for b in range(NBUF):
                descs[b].wait()
                pltpu.sync_copy(rows[b], shared.at[idx_d.at[j0 + b]], add=True)
            return carry

        lax.fori_loop(0, CH_W // NBUF, group, 0)
        plsc.subcore_barrier()
        pltpu.sync_copy(shared.at[pl.ds(sid * ROWS_W, ROWS_W)],
                        out_ref.at[pl.ds(out_base + sid * ROWS_W, ROWS_W)])
        plsc.subcore_barrier()

    phase(dlo2, 0)
    phase(dhi2, NHALF)


@functools.lru_cache(maxsize=None)
def _mesh1():
    return plsc.VectorSubcoreMesh(core_axis_name="c", subcore_axis_name="s",
                                  num_cores=1, num_subcores=NS)


@functools.lru_cache(maxsize=None)
def _k3_kernel():
    return pl.kernel(
        _k3_body,
        out_type=jax.ShapeDtypeStruct((NP, HH), F32),
        mesh=_mesh1(),
        scratch_types=[
            pltpu.VMEM((CH_W, 128), jnp.int32),
            pltpu.VMEM((CH_W, 128), jnp.int32),
            pltpu.VMEM((128, HH), F32),
            pltpu.VMEM((128, HH), F32),
            pltpu.VMEM((128, HH), F32),
            pltpu.VMEM((128, HH), F32),
            pltpu.SemaphoreType.DMA,
            pltpu.SemaphoreType.DMA,
            pltpu.SemaphoreType.DMA,
            pltpu.SemaphoreType.DMA,
            pltpu.VMEM_SHARED((SLAB, HH), F32),
        ],
    )


def _k3_call(src2, dlo2, dhi2, x1h, zrows):
    return _k3_kernel()(src2, dlo2, dhi2, x1h, zrows)


def _k4_body(agg_a, agg_b, gh, nf, wi, bi, h_o, sums_o):
    i = pl.program_id(0)
    agg = jnp.concatenate([agg_a[...], agg_b[...]], axis=1)
    gi = _dot_nt(agg, wi[...]) + bi[...]
    ghv = gh[...]
    r = jax.nn.sigmoid(gi[:, 0:H] + ghv[:, 0:H])
    z = jax.nn.sigmoid(gi[:, H:2 * H] + ghv[:, H:2 * H])
    n = jnp.tanh(gi[:, 2 * H:] + r * ghv[:, 2 * H:])
    h = (1.0 - z) * n + z * nf[...]
    h_o[...] = h
    row = lax.broadcasted_iota(jnp.int32, (BLK, 1), 0) + i * BLK
    hm = jnp.where(row < N, h, 0.0)
    s = jnp.sum(hm, axis=0, keepdims=True)
    ss = jnp.sum(hm * hm, axis=0, keepdims=True)
    pack = jnp.concatenate([s, ss, jnp.zeros((6, H), dtype=F32)], axis=0)

    @pl.when(i == 0)
    def _():
        sums_o[...] = pack

    @pl.when(i > 0)
    def _():
        sums_o[...] = sums_o[...] + pack


def _k4_call(agg_a, agg_b, gh, nf, Wi, bi_row):
    grid = (NP // BLK,)
    return pl.pallas_call(
        _k4_body,
        grid=grid,
        in_specs=[
            pl.BlockSpec((BLK, HH), lambda i: (i, 0)),
            pl.BlockSpec((BLK, HH), lambda i: (i, 0)),
            pl.BlockSpec((BLK, 3 * H), lambda i: (i, 0)),
            pl.BlockSpec((BLK, H), lambda i: (i, 0)),
            pl.BlockSpec((3 * H, H), lambda i: (0, 0)),
            pl.BlockSpec((1, 3 * H), lambda i: (0, 0)),
        ],
        out_specs=[
            pl.BlockSpec((BLK, H), lambda i: (i, 0)),
            pl.BlockSpec((8, H), lambda i: (0, 0)),
        ],
        out_shape=[
            jax.ShapeDtypeStruct((NP, H), F32),
            jax.ShapeDtypeStruct((8, H), F32),
        ],
    )(agg_a, agg_b, gh, nf, Wi, bi_row)


def _k5_body(e1_idx, rel_idx, h, rel_emb, he, re, idx_v, rows_v, sem):
    wid = lax.axis_index("s") * NC + lax.axis_index("c")
    per = B // NW
    base = wid * per
    pltpu.sync_copy(e1_idx.at[pl.ds(base, per)], idx_v)
    pltpu.async_copy(h.at[idx_v], rows_v, sem).wait()
    pltpu.sync_copy(rows_v, he.at[pl.ds(base, per)])
    pltpu.sync_copy(rel_idx.at[pl.ds(base, per)], idx_v)
    pltpu.async_copy(rel_emb.at[idx_v], rows_v, sem).wait()
    pltpu.sync_copy(rows_v, re.at[pl.ds(base, per)])


@functools.lru_cache(maxsize=None)
def _k5_kernel():
    return pl.kernel(
        _k5_body,
        out_type=[
            jax.ShapeDtypeStruct((B, H), F32),
            jax.ShapeDtypeStruct((B, H), F32),
        ],
        mesh=_mesh(),
        scratch_types=[
            pltpu.VMEM((B // NW,), jnp.int32),
            pltpu.VMEM((B // NW, H), F32),
            pltpu.SemaphoreType.DMA,
        ],
    )


def _k5_call(e1_idx, rel_idx, h, rel_emb):
    return _k5_kernel()(e1_idx, rel_idx, h, rel_emb)


def _k6_body(he, re, sums, gamma, beta, h, e2, logits_o, loss_o):
    i = pl.program_id(0)
    ng = pl.num_programs(0)
    inv_n = 1.0 / N
    mean = sums[0:1, :] * inv_n
    var = sums[1:2, :] * inv_n - mean * mean
    sc = lax.rsqrt(var + 1e-5) * gamma[...]
    q = ((he[...] - mean) * sc + beta[...]) * re[...]
    hb = (h[...] - mean) * sc + beta[...]
    lg = jax.nn.sigmoid(_dot_nt(q, hb))
    logits_o[...] = lg
    p = jnp.clip(lg, 1e-7, 1.0 - 1e-7)
    e2v = e2[...]
    col = lax.broadcasted_iota(jnp.int32, (1, BLK), 1) + i * BLK
    term = e2v * jnp.log(p) + (1.0 - e2v) * jnp.log(1.0 - p)
    part = jnp.sum(jnp.where(col < N, term, 0.0))

    @pl.when(i == 0)
    def _():
        loss_o[0, 0] = part

    @pl.when(i > 0)
    def _():
        loss_o[0, 0] = loss_o[0, 0] + part

    @pl.when(i == ng - 1)
    def _():
        loss_o[0, 0] = loss_o[0, 0] * (-1.0 / (B * N))


def _k6_call(he, re, sums, gamma_row, beta_row, h, e2_pad):
    grid = (NP // BLK,)
    return pl.pallas_call(
        _k6_body,
        grid=grid,
        in_specs=[
            pl.BlockSpec((B, H), lambda i: (0, 0)),
            pl.BlockSpec((B, H), lambda i: (0, 0)),
            pl.BlockSpec((8, H), lambda i: (0, 0)),
            pl.BlockSpec((1, H), lambda i: (0, 0)),
            pl.BlockSpec((1, H), lambda i: (0, 0)),
            pl.BlockSpec((BLK, H), lambda i: (i, 0)),
            pl.BlockSpec((B, BLK), lambda i: (0, i)),
        ],
        out_specs=[
            pl.BlockSpec((B, BLK), lambda i: (0, i)),
            pl.BlockSpec(memory_space=pltpu.SMEM),
        ],
        out_shape=[
            jax.ShapeDtypeStruct((B, NP), F32),
            jax.ShapeDtypeStruct((1, 1), F32),
        ],
    )(he, re, sums, gamma_row, beta_row, h, e2_pad)


def kernel(node_token_idx, edge_index, e1, rel, e2_multi, word_emb,
           W_msg, Wi, Wh, bi, bh, bn_gamma, bn_beta, rel_emb):
    tok_flat = jnp.concatenate(
        [node_token_idx.reshape(-1),
         jnp.zeros((TOK_PAD - N * T,), jnp.int32)])
    src2 = jnp.concatenate(
        [edge_index[0], jnp.zeros((E_PAD - E,), jnp.int32)]).reshape(-1, 128)
    dst2 = jnp.concatenate(
        [edge_index[1], jnp.full((E_PAD - E,), NP, jnp.int32)]).reshape(-1, 128)

    tok_rows = _k1_call(tok_flat, word_emb)
    tok2 = tok_rows.reshape(NP, 4 * H)

    nf, x1a, x1b, gh, dlo2, dhi2 = _k2_call(tok2, W_msg, Wh, bh.reshape(1, 3 * H), dst2)

    zrows = jnp.zeros((ROWS_Z, HH), F32)
    agg_a = _k3_call(src2, dlo2, dhi2, x1a, zrows)
    agg_b = _k3_call(src2, dlo2, dhi2, x1b, zrows)

    h, sums = _k4_call(agg_a, agg_b, gh, nf, Wi, bi.reshape(1, 3 * H))

    he, re = _k5_call(e1[:, 0], rel[:, 0], h, rel_emb)

    logits_pad, loss = _k6_call(he, re, sums, bn_gamma.reshape(1, H),
                                bn_beta.reshape(1, H), h, e2_multi)
    return logits_pad[:, :N], loss[0, 0]


# speedup vs baseline: 1.6844x; 1.6844x over previous
import functools

import jax
import jax.numpy as jnp
from jax import lax
from jax.experimental import pallas as pl
from jax.experimental.pallas import tpu as pltpu
from jax.experimental.pallas import tpu_sc as plsc

N = 10000
E = 160000
H = 256
B = 1024
R = 64
V = 50000
T = 4

NC = 2
NS = 16
NW = NC * NS

NP = 10240
TOK_PAD = NP * T
E_PAD = 163840
ROWS_SC = NP // NS
HH = H // 2
HQ = H // 4
BLK = 1024

F32 = jnp.float32


def _dot_nt(a, b):
    return lax.dot_general(a, b, (((1,), (1,)), ((), ())),
                           preferred_element_type=F32)


def _dot_nn(a, b):
    return lax.dot_general(a, b, (((1,), (0,)), ((), ())),
                           preferred_element_type=F32)


@functools.lru_cache(maxsize=None)
def _mesh():
    return plsc.VectorSubcoreMesh(core_axis_name="c", subcore_axis_name="s",
                                  num_cores=NC, num_subcores=NS)


def _k1_body(tok_idx, wemb, out, idx_v, rows_v, sem):
    wid = lax.axis_index("s") * NC + lax.axis_index("c")
    per = TOK_PAD // NW
    base = wid * per

    def chunk(j, carry):
        off = base + j * 128
        pltpu.sync_copy(tok_idx.at[pl.ds(off, 128)], idx_v)
        pltpu.async_copy(wemb.at[idx_v], rows_v, sem).wait()
        pltpu.sync_copy(rows_v, out.at[pl.ds(off, 128)])
        return carry

    lax.fori_loop(0, per // 128, chunk, 0)


@functools.lru_cache(maxsize=None)
def _k1_kernel():
    return pl.kernel(
        _k1_body,
        out_type=jax.ShapeDtypeStruct((TOK_PAD, H), F32),
        mesh=_mesh(),
        scratch_types=[
            pltpu.VMEM((128,), jnp.int32),
            pltpu.VMEM((128, H), F32),
            pltpu.SemaphoreType.DMA,
        ],
    )


def _k1_call(tok_idx, wemb):
    return _k1_kernel()(tok_idx, wemb)


def _k2_body(tok2, wmsg, wh, bh, dst2, nf_o, x1a_o, x1b_o, gh_o, dlo_o, dhi_o):
    t = tok2[...]
    nf = (t[:, 0:H] + t[:, H:2 * H] + t[:, 2 * H:3 * H] + t[:, 3 * H:4 * H]) * 0.25
    x1 = _dot_nn(nf, wmsg[...])
    x1a_o[...] = x1[:, :HH]
    x1b_o[...] = x1[:, HH:]
    gh_o[...] = _dot_nt(nf, wh[...]) + bh[...]
    nf_o[...] = nf
    d = dst2[...]
    dlo_o[...] = jnp.where(d < NHALF, d, NHALF)
    dhi_o[...] = jnp.where(d >= NHALF, d - NHALF, NHALF)


def _k2_call(tok2, W_msg, Wh, bh_row, dst2):
    grid = (NP // BLK,)
    return pl.pallas_call(
        _k2_body,
        grid=grid,
        in_specs=[
            pl.BlockSpec((BLK, 4 * H), lambda i: (i, 0)),
            pl.BlockSpec((H, H), lambda i: (0, 0)),
            pl.BlockSpec((3 * H, H), lambda i: (0, 0)),
            pl.BlockSpec((1, 3 * H), lambda i: (0, 0)),
            pl.BlockSpec((E_PAD // 128 // (NP // BLK), 128), lambda i: (i, 0)),
        ],
        out_specs=[
            pl.BlockSpec((BLK, H), lambda i: (i, 0)),
            pl.BlockSpec((BLK, HH), lambda i: (i, 0)),
            pl.BlockSpec((BLK, HH), lambda i: (i, 0)),
            pl.BlockSpec((BLK, 3 * H), lambda i: (i, 0)),
            pl.BlockSpec((E_PAD // 128 // (NP // BLK), 128), lambda i: (i, 0)),
            pl.BlockSpec((E_PAD // 128 // (NP // BLK), 128), lambda i: (i, 0)),
        ],
        out_shape=[
            jax.ShapeDtypeStruct((NP, H), F32),
            jax.ShapeDtypeStruct((NP, HH), F32),
            jax.ShapeDtypeStruct((NP, HH), F32),
            jax.ShapeDtypeStruct((NP, 3 * H), F32),
            jax.ShapeDtypeStruct((E_PAD // 128, 128), jnp.int32),
            jax.ShapeDtypeStruct((E_PAD // 128, 128), jnp.int32),
        ],
    )(tok2, W_msg, Wh, bh_row, dst2)


NBUF = 4
CH_W = (E_PAD // NS) // 128


NHALF = NP // 2
SLAB = NHALF + 128
ROWS_Z = SLAB // NS
ROWS_W = NHALF // NS


def _k3_body(src2, dlo2, dhi2, x1a, x1b, zrows, agg_a, agg_b,
             idx_s, idx_d, rows0, rows1, rows2, rows3,
             semg0, semg1, semg2, semg3, sems0, sems1, sems2, sems3, shared):
    cid = lax.axis_index("c")
    sid = lax.axis_index("s")
    rows = (rows0, rows1, rows2, rows3)
    semg = (semg0, semg1, semg2, semg3)
    sems = (sems0, sems1, sems2, sems3)

    row_base = sid * CH_W
    pltpu.sync_copy(src2.at[pl.ds(row_base, CH_W)], idx_s)

    def phase(table, dst2, out_ref, out_base):
        pltpu.sync_copy(dst2.at[pl.ds(row_base, CH_W)], idx_d)
        pltpu.sync_copy(zrows, shared.at[pl.ds(sid * ROWS_Z, ROWS_Z)])
        plsc.subcore_barrier()

        def group(g, carry):
            j0 = g * NBUF
            for b in range(NBUF):
                @pl.when(g > 0)
                def _():
                    pltpu.make_async_copy(
                        rows[b], shared.at[idx_d.at[j0 + b]], sems[b]).wait()
                pltpu.async_copy(table.at[idx_s.at[j0 + b]], rows[b], semg[b])
            for b in range(NBUF):
                pltpu.make_async_copy(
                    table.at[idx_s.at[j0 + b]], rows[b], semg[b]).wait()
                pltpu.async_copy(
                    rows[b], shared.at[idx_d.at[j0 + b]], sems[b], add=True)
            return carry

        lax.fori_loop(0, CH_W // NBUF, group, 0)
        for b in range(NBUF):
            pltpu.make_async_copy(
                rows[b], shared.at[idx_d.at[CH_W - NBUF + b]], sems[b]).wait()
        plsc.subcore_barrier()
        pltpu.sync_copy(shared.at[pl.ds(sid * ROWS_W, ROWS_W)],
                        out_ref.at[pl.ds(out_base + sid * ROWS_W, ROWS_W)])
        plsc.subcore_barrier()

    @pl.when(cid == 0)
    def _():
        phase(x1a, dlo2, agg_a, 0)
        phase(x1a, dhi2, agg_a, NHALF)

    @pl.when(cid == 1)
    def _():
        phase(x1b, dlo2, agg_b, 0)
        phase(x1b, dhi2, agg_b, NHALF)


@functools.lru_cache(maxsize=None)
def _k3_kernel():
    return pl.kernel(
        _k3_body,
        out_type=[
            jax.ShapeDtypeStruct((NP, HH), F32),
            jax.ShapeDtypeStruct((NP, HH), F32),
        ],
        mesh=_mesh(),
        scratch_types=[
            pltpu.VMEM((CH_W, 128), jnp.int32),
            pltpu.VMEM((CH_W, 128), jnp.int32),
            pltpu.VMEM((128, HH), F32),
            pltpu.VMEM((128, HH), F32),
            pltpu.VMEM((128, HH), F32),
            pltpu.VMEM((128, HH), F32),
            pltpu.SemaphoreType.DMA,
            pltpu.SemaphoreType.DMA,
            pltpu.SemaphoreType.DMA,
            pltpu.SemaphoreType.DMA,
            pltpu.SemaphoreType.DMA,
            pltpu.SemaphoreType.DMA,
            pltpu.SemaphoreType.DMA,
            pltpu.SemaphoreType.DMA,
            pltpu.VMEM_SHARED((SLAB, HH), F32),
        ],
    )


def _k3_call(src2, dlo2, dhi2, x1a, x1b, zrows):
    return _k3_kernel()(src2, dlo2, dhi2, x1a, x1b, zrows)


def _k4_body(agg_a, agg_b, gh, nf, wi, bi, h_o, sums_o):
    i = pl.program_id(0)
    agg = jnp.concatenate([agg_a[...], agg_b[...]], axis=1)
    gi = _dot_nt(agg, wi[...]) + bi[...]
    ghv = gh[...]
    r = jax.nn.sigmoid(gi[:, 0:H] + ghv[:, 0:H])
    z = jax.nn.sigmoid(gi[:, H:2 * H] + ghv[:, H:2 * H])
    n = jnp.tanh(gi[:, 2 * H:] + r * ghv[:, 2 * H:])
    h = (1.0 - z) * n + z * nf[...]
    h_o[...] = h
    row = lax.broadcasted_iota(jnp.int32, (BLK, 1), 0) + i * BLK
    hm = jnp.where(row < N, h, 0.0)
    s = jnp.sum(hm, axis=0, keepdims=True)
    ss = jnp.sum(hm * hm, axis=0, keepdims=True)
    pack = jnp.concatenate([s, ss, jnp.zeros((6, H), dtype=F32)], axis=0)

    @pl.when(i == 0)
    def _():
        sums_o[...] = pack

    @pl.when(i > 0)
    def _():
        sums_o[...] = sums_o[...] + pack


def _k4_call(agg_a, agg_b, gh, nf, Wi, bi_row):
    grid = (NP // BLK,)
    return pl.pallas_call(
        _k4_body,
        grid=grid,
        in_specs=[
            pl.BlockSpec((BLK, HH), lambda i: (i, 0)),
            pl.BlockSpec((BLK, HH), lambda i: (i, 0)),
            pl.BlockSpec((BLK, 3 * H), lambda i: (i, 0)),
            pl.BlockSpec((BLK, H), lambda i: (i, 0)),
            pl.BlockSpec((3 * H, H), lambda i: (0, 0)),
            pl.BlockSpec((1, 3 * H), lambda i: (0, 0)),
        ],
        out_specs=[
            pl.BlockSpec((BLK, H), lambda i: (i, 0)),
            pl.BlockSpec((8, H), lambda i: (0, 0)),
        ],
        out_shape=[
            jax.ShapeDtypeStruct((NP, H), F32),
            jax.ShapeDtypeStruct((8, H), F32),
        ],
    )(agg_a, agg_b, gh, nf, Wi, bi_row)


def _k5_body(e1_idx, rel_idx, h, rel_emb, he, re, idx_v, rows_v, sem):
    wid = lax.axis_index("s") * NC + lax.axis_index("c")
    per = B // NW
    base = wid * per
    pltpu.sync_copy(e1_idx.at[pl.ds(base, per)], idx_v)
    pltpu.async_copy(h.at[idx_v], rows_v, sem).wait()
    pltpu.sync_copy(rows_v, he.at[pl.ds(base, per)])
    pltpu.sync_copy(rel_idx.at[pl.ds(base, per)], idx_v)
    pltpu.async_copy(rel_emb.at[idx_v], rows_v, sem).wait()
    pltpu.sync_copy(rows_v, re.at[pl.ds(base, per)])


@functools.lru_cache(maxsize=None)
def _k5_kernel():
    return pl.kernel(
        _k5_body,
        out_type=[
            jax.ShapeDtypeStruct((B, H), F32),
            jax.ShapeDtypeStruct((B, H), F32),
        ],
        mesh=_mesh(),
        scratch_types=[
            pltpu.VMEM((B // NW,), jnp.int32),
            pltpu.VMEM((B // NW, H), F32),
            pltpu.SemaphoreType.DMA,
        ],
    )


def _k5_call(e1_idx, rel_idx, h, rel_emb):
    return _k5_kernel()(e1_idx, rel_idx, h, rel_emb)


def _k6_body(he, re, sums, gamma, beta, h, e2, logits_o, loss_o):
    i = pl.program_id(0)
    ng = pl.num_programs(0)
    inv_n = 1.0 / N
    mean = sums[0:1, :] * inv_n
    var = sums[1:2, :] * inv_n - mean * mean
    sc = lax.rsqrt(var + 1e-5) * gamma[...]
    q = ((he[...] - mean) * sc + beta[...]) * re[...]
    hb = (h[...] - mean) * sc + beta[...]
    lg = jax.nn.sigmoid(_dot_nt(q, hb))
    logits_o[...] = lg
    p = jnp.clip(lg, 1e-7, 1.0 - 1e-7)
    e2v = e2[...]
    col = lax.broadcasted_iota(jnp.int32, (1, BLK), 1) + i * BLK
    term = e2v * jnp.log(p) + (1.0 - e2v) * jnp.log(1.0 - p)
    part = jnp.sum(jnp.where(col < N, term, 0.0))

    @pl.when(i == 0)
    def _():
        loss_o[0, 0] = part

    @pl.when(i > 0)
    def _():
        loss_o[0, 0] = loss_o[0, 0] + part

    @pl.when(i == ng - 1)
    def _():
        loss_o[0, 0] = loss_o[0, 0] * (-1.0 / (B * N))


def _k6_call(he, re, sums, gamma_row, beta_row, h, e2_pad):
    grid = (NP // BLK,)
    return pl.pallas_call(
        _k6_body,
        grid=grid,
        in_specs=[
            pl.BlockSpec((B, H), lambda i: (0, 0)),
            pl.BlockSpec((B, H), lambda i: (0, 0)),
            pl.BlockSpec((8, H), lambda i: (0, 0)),
            pl.BlockSpec((1, H), lambda i: (0, 0)),
            pl.BlockSpec((1, H), lambda i: (0, 0)),
            pl.BlockSpec((BLK, H), lambda i: (i, 0)),
            pl.BlockSpec((B, BLK), lambda i: (0, i)),
        ],
        out_specs=[
            pl.BlockSpec((B, BLK), lambda i: (0, i)),
            pl.BlockSpec(memory_space=pltpu.SMEM),
        ],
        out_shape=[
            jax.ShapeDtypeStruct((B, NP), F32),
            jax.ShapeDtypeStruct((1, 1), F32),
        ],
    )(he, re, sums, gamma_row, beta_row, h, e2_pad)


def kernel(node_token_idx, edge_index, e1, rel, e2_multi, word_emb,
           W_msg, Wi, Wh, bi, bh, bn_gamma, bn_beta, rel_emb):
    tok_flat = jnp.concatenate(
        [node_token_idx.reshape(-1),
         jnp.zeros((TOK_PAD - N * T,), jnp.int32)])
    src2 = jnp.concatenate(
        [edge_index[0], jnp.zeros((E_PAD - E,), jnp.int32)]).reshape(-1, 128)
    dst2 = jnp.concatenate(
        [edge_index[1], jnp.full((E_PAD - E,), NP, jnp.int32)]).reshape(-1, 128)

    tok_rows = _k1_call(tok_flat, word_emb)
    tok2 = tok_rows.reshape(NP, 4 * H)

    nf, x1a, x1b, gh, dlo2, dhi2 = _k2_call(tok2, W_msg, Wh, bh.reshape(1, 3 * H), dst2)

    zrows = jnp.zeros((ROWS_Z, HH), F32)
    agg_a, agg_b = _k3_call(src2, dlo2, dhi2, x1a, x1b, zrows)

    h, sums = _k4_call(agg_a, agg_b, gh, nf, Wi, bi.reshape(1, 3 * H))

    he, re = _k5_call(e1[:, 0], rel[:, 0], h, rel_emb)

    logits_pad, loss = _k6_call(he, re, sums, bn_gamma.reshape(1, H),
                                bn_beta.reshape(1, H), h, e2_multi)
    return logits_pad[:, :N], loss[0, 0]


# speedup vs baseline: 2.3071x; 1.3697x over previous
import functools

import jax
import jax.numpy as jnp
from jax import lax
from jax.experimental import pallas as pl
from jax.experimental.pallas import tpu as pltpu
from jax.experimental.pallas import tpu_sc as plsc

N = 10000
E = 160000
H = 256
B = 1024
R = 64
V = 50000
T = 4

NC = 2
NS = 16
NW = NC * NS

NP = 10240
TOK_PAD = NP * T
E_PAD = 163840
ROWS_SC = NP // NS
HH = H // 2
HQ = H // 4
BLK = 1024

F32 = jnp.float32


def _dot_nt(a, b):
    return lax.dot_general(a, b, (((1,), (1,)), ((), ())),
                           preferred_element_type=F32)


def _dot_nn(a, b):
    return lax.dot_general(a, b, (((1,), (0,)), ((), ())),
                           preferred_element_type=F32)


@functools.lru_cache(maxsize=None)
def _mesh():
    return plsc.VectorSubcoreMesh(core_axis_name="c", subcore_axis_name="s",
                                  num_cores=NC, num_subcores=NS)


def _k1_body(tok_idx, wemb, out, idx_v, rows_v, sem):
    wid = lax.axis_index("s") * NC + lax.axis_index("c")
    per = TOK_PAD // NW
    base = wid * per

    def chunk(j, carry):
        off = base + j * 128
        pltpu.sync_copy(tok_idx.at[pl.ds(off, 128)], idx_v)
        pltpu.async_copy(wemb.at[idx_v], rows_v, sem).wait()
        pltpu.sync_copy(rows_v, out.at[pl.ds(off, 128)])
        return carry

    lax.fori_loop(0, per // 128, chunk, 0)


@functools.lru_cache(maxsize=None)
def _k1_kernel():
    return pl.kernel(
        _k1_body,
        out_type=jax.ShapeDtypeStruct((TOK_PAD, H), F32),
        mesh=_mesh(),
        scratch_types=[
            pltpu.VMEM((128,), jnp.int32),
            pltpu.VMEM((128, H), F32),
            pltpu.SemaphoreType.DMA,
        ],
    )


def _k1_call(tok_idx, wemb):
    return _k1_kernel()(tok_idx, wemb)


def _k2_body(tok2, wmsg, wh, bh, nf_o, x1a_o, x1b_o, gh_o):
    t = tok2[...]
    nf = (t[:, 0:H] + t[:, H:2 * H] + t[:, 2 * H:3 * H] + t[:, 3 * H:4 * H]) * 0.25
    x1 = _dot_nn(nf, wmsg[...])
    x1a_o[...] = x1[:, :HH]
    x1b_o[...] = x1[:, HH:]
    gh_o[...] = _dot_nt(nf, wh[...]) + bh[...]
    nf_o[...] = nf


def _k2_call(tok2, W_msg, Wh, bh_row):
    grid = (NP // BLK,)
    return pl.pallas_call(
        _k2_body,
        grid=grid,
        in_specs=[
            pl.BlockSpec((BLK, 4 * H), lambda i: (i, 0)),
            pl.BlockSpec((H, H), lambda i: (0, 0)),
            pl.BlockSpec((3 * H, H), lambda i: (0, 0)),
            pl.BlockSpec((1, 3 * H), lambda i: (0, 0)),
        ],
        out_specs=[
            pl.BlockSpec((BLK, H), lambda i: (i, 0)),
            pl.BlockSpec((BLK, HH), lambda i: (i, 0)),
            pl.BlockSpec((BLK, HH), lambda i: (i, 0)),
            pl.BlockSpec((BLK, 3 * H), lambda i: (i, 0)),
        ],
        out_shape=[
            jax.ShapeDtypeStruct((NP, H), F32),
            jax.ShapeDtypeStruct((NP, HH), F32),
            jax.ShapeDtypeStruct((NP, HH), F32),
            jax.ShapeDtypeStruct((NP, 3 * H), F32),
        ],
    )(tok2, W_msg, Wh, bh_row)


NBUF = 1
CH_W = (E_PAD // NS) // 128


SLAB = NP + 128
ROWS_Z = SLAB // NS
ROWS_W = NP // NS


def _k3_body(src2, dst2, x1a, x1b, zrows, agg_a, agg_b,
             idx_s, idx_d, ib0, rows0, semg, sems, shared):
    cid = lax.axis_index("c")
    sid = lax.axis_index("s")

    row_base = sid * CH_W
    pltpu.sync_copy(src2.at[pl.ds(row_base, CH_W)], idx_s)
    pltpu.sync_copy(dst2.at[pl.ds(row_base, CH_W)], idx_d)
    pltpu.sync_copy(zrows, shared.at[pl.ds(sid * ROWS_Z, ROWS_Z)])
    plsc.subcore_barrier()

    def run(table, out_ref):
        def step(j, carry):
            @pl.when(j > 0)
            def _():
                pltpu.make_async_copy(rows0, shared.at[ib0], sems).wait()
            for l in range(8):
                ib0[pl.ds(l * 16, 16)] = idx_d[j, pl.ds(l * 16, 16)]
            pltpu.async_copy(table.at[idx_s.at[j]], rows0, semg).wait()
            pltpu.async_copy(rows0, shared.at[ib0], sems, add=True)
            return carry

        lax.fori_loop(0, CH_W, step, 0)
        pltpu.make_async_copy(rows0, shared.at[ib0], sems).wait()
        plsc.subcore_barrier()
        pltpu.sync_copy(shared.at[pl.ds(sid * ROWS_W, ROWS_W)],
                        out_ref.at[pl.ds(sid * ROWS_W, ROWS_W)])

    @pl.when(cid == 0)
    def _():
        run(x1a, agg_a)

    @pl.when(cid == 1)
    def _():
        run(x1b, agg_b)


@functools.lru_cache(maxsize=None)
def _k3_kernel():
    return pl.kernel(
        _k3_body,
        out_type=[
            jax.ShapeDtypeStruct((NP, HH), F32),
            jax.ShapeDtypeStruct((NP, HH), F32),
        ],
        mesh=_mesh(),
        scratch_types=[
            pltpu.VMEM((CH_W, 128), jnp.int32),
            pltpu.VMEM((CH_W, 128), jnp.int32),
            pltpu.VMEM((128,), jnp.int32),
            pltpu.VMEM((128, HH), F32),
            pltpu.SemaphoreType.DMA,
            pltpu.SemaphoreType.DMA,
            pltpu.VMEM_SHARED((SLAB, HH), F32),
        ],
    )


def _k3_call(src2, dst2, x1a, x1b, zrows):
    return _k3_kernel()(src2, dst2, x1a, x1b, zrows)


def _k4_body(agg_a, agg_b, gh, nf, wi, bi, h_o, sums_o):
    i = pl.program_id(0)
    agg = jnp.concatenate([agg_a[...], agg_b[...]], axis=1)
    gi = _dot_nt(agg, wi[...]) + bi[...]
    ghv = gh[...]
    r = jax.nn.sigmoid(gi[:, 0:H] + ghv[:, 0:H])
    z = jax.nn.sigmoid(gi[:, H:2 * H] + ghv[:, H:2 * H])
    n = jnp.tanh(gi[:, 2 * H:] + r * ghv[:, 2 * H:])
    h = (1.0 - z) * n + z * nf[...]
    h_o[...] = h
    row = lax.broadcasted_iota(jnp.int32, (BLK, 1), 0) + i * BLK
    hm = jnp.where(row < N, h, 0.0)
    s = jnp.sum(hm, axis=0, keepdims=True)
    ss = jnp.sum(hm * hm, axis=0, keepdims=True)
    pack = jnp.concatenate([s, ss, jnp.zeros((6, H), dtype=F32)], axis=0)

    @pl.when(i == 0)
    def _():
        sums_o[...] = pack

    @pl.when(i > 0)
    def _():
        sums_o[...] = sums_o[...] + pack


def _k4_call(agg_a, agg_b, gh, nf, Wi, bi_row):
    grid = (NP // BLK,)
    return pl.pallas_call(
        _k4_body,
        grid=grid,
        in_specs=[
            pl.BlockSpec((BLK, HH), lambda i: (i, 0)),
            pl.BlockSpec((BLK, HH), lambda i: (i, 0)),
            pl.BlockSpec((BLK, 3 * H), lambda i: (i, 0)),
            pl.BlockSpec((BLK, H), lambda i: (i, 0)),
            pl.BlockSpec((3 * H, H), lambda i: (0, 0)),
            pl.BlockSpec((1, 3 * H), lambda i: (0, 0)),
        ],
        out_specs=[
            pl.BlockSpec((BLK, H), lambda i: (i, 0)),
            pl.BlockSpec((8, H), lambda i: (0, 0)),
        ],
        out_shape=[
            jax.ShapeDtypeStruct((NP, H), F32),
            jax.ShapeDtypeStruct((8, H), F32),
        ],
    )(agg_a, agg_b, gh, nf, Wi, bi_row)


def _k5_body(e1_idx, rel_idx, h, rel_emb, he, re, idx_v, rows_v, sem):
    wid = lax.axis_index("s") * NC + lax.axis_index("c")
    per = B // NW
    base = wid * per
    pltpu.sync_copy(e1_idx.at[pl.ds(base, per)], idx_v)
    pltpu.async_copy(h.at[idx_v], rows_v, sem).wait()
    pltpu.sync_copy(rows_v, he.at[pl.ds(base, per)])
    pltpu.sync_copy(rel_idx.at[pl.ds(base, per)], idx_v)
    pltpu.async_copy(rel_emb.at[idx_v], rows_v, sem).wait()
    pltpu.sync_copy(rows_v, re.at[pl.ds(base, per)])


@functools.lru_cache(maxsize=None)
def _k5_kernel():
    return pl.kernel(
        _k5_body,
        out_type=[
            jax.ShapeDtypeStruct((B, H), F32),
            jax.ShapeDtypeStruct((B, H), F32),
        ],
        mesh=_mesh(),
        scratch_types=[
            pltpu.VMEM((B // NW,), jnp.int32),
            pltpu.VMEM((B // NW, H), F32),
            pltpu.SemaphoreType.DMA,
        ],
    )


def _k5_call(e1_idx, rel_idx, h, rel_emb):
    return _k5_kernel()(e1_idx, rel_idx, h, rel_emb)


def _k6_body(he, re, sums, gamma, beta, h, e2, logits_o, loss_o):
    i = pl.program_id(0)
    ng = pl.num_programs(0)
    inv_n = 1.0 / N
    mean = sums[0:1, :] * inv_n
    var = sums[1:2, :] * inv_n - mean * mean
    sc = lax.rsqrt(var + 1e-5) * gamma[...]
    q = ((he[...] - mean) * sc + beta[...]) * re[...]
    hb = (h[...] - mean) * sc + beta[...]
    lg = jax.nn.sigmoid(_dot_nt(q, hb))
    logits_o[...] = lg
    p = jnp.clip(lg, 1e-7, 1.0 - 1e-7)
    e2v = e2[...]
    col = lax.broadcasted_iota(jnp.int32, (1, BLK), 1) + i * BLK
    term = e2v * jnp.log(p) + (1.0 - e2v) * jnp.log(1.0 - p)
    part = jnp.sum(jnp.where(col < N, term, 0.0))

    @pl.when(i == 0)
    def _():
        loss_o[0, 0] = part

    @pl.when(i > 0)
    def _():
        loss_o[0, 0] = loss_o[0, 0] + part

    @pl.when(i == ng - 1)
    def _():
        loss_o[0, 0] = loss_o[0, 0] * (-1.0 / (B * N))


def _k6_call(he, re, sums, gamma_row, beta_row, h, e2_pad):
    grid = (NP // BLK,)
    return pl.pallas_call(
        _k6_body,
        grid=grid,
        in_specs=[
            pl.BlockSpec((B, H), lambda i: (0, 0)),
            pl.BlockSpec((B, H), lambda i: (0, 0)),
            pl.BlockSpec((8, H), lambda i: (0, 0)),
            pl.BlockSpec((1, H), lambda i: (0, 0)),
            pl.BlockSpec((1, H), lambda i: (0, 0)),
            pl.BlockSpec((BLK, H), lambda i: (i, 0)),
            pl.BlockSpec((B, BLK), lambda i: (0, i)),
        ],
        out_specs=[
            pl.BlockSpec((B, BLK), lambda i: (0, i)),
            pl.BlockSpec(memory_space=pltpu.SMEM),
        ],
        out_shape=[
            jax.ShapeDtypeStruct((B, NP), F32),
            jax.ShapeDtypeStruct((1, 1), F32),
        ],
    )(he, re, sums, gamma_row, beta_row, h, e2_pad)


def kernel(node_token_idx, edge_index, e1, rel, e2_multi, word_emb,
           W_msg, Wi, Wh, bi, bh, bn_gamma, bn_beta, rel_emb):
    tok_flat = jnp.concatenate(
        [node_token_idx.reshape(-1),
         jnp.zeros((TOK_PAD - N * T,), jnp.int32)])
    src2 = jnp.concatenate(
        [edge_index[0], jnp.zeros((E_PAD - E,), jnp.int32)]).reshape(-1, 128)
    dst2 = jnp.concatenate(
        [edge_index[1], jnp.full((E_PAD - E,), NP, jnp.int32)]).reshape(-1, 128)

    tok_rows = _k1_call(tok_flat, word_emb)
    tok2 = tok_rows.reshape(NP, 4 * H)

    nf, x1a, x1b, gh = _k2_call(tok2, W_msg, Wh, bh.reshape(1, 3 * H))

    zrows = jnp.zeros((ROWS_Z, HH), F32)
    agg_a, agg_b = _k3_call(src2, dst2, x1a, x1b, zrows)

    h, sums = _k4_call(agg_a, agg_b, gh, nf, Wi, bi.reshape(1, 3 * H))

    he, re = _k5_call(e1[:, 0], rel[:, 0], h, rel_emb)

    logits_pad, loss = _k6_call(he, re, sums, bn_gamma.reshape(1, H),
                                bn_beta.reshape(1, H), h, e2_multi)
    return logits_pad[:, :N], loss[0, 0]


# speedup vs baseline: 2.5299x; 1.0966x over previous
import functools

import jax
import jax.numpy as jnp
from jax import lax
from jax.experimental import pallas as pl
from jax.experimental.pallas import tpu as pltpu
from jax.experimental.pallas import tpu_sc as plsc

N = 10000
E = 160000
H = 256
B = 1024
R = 64
V = 50000
T = 4

NC = 2
NS = 16
NW = NC * NS

NP = 10240
TOK_PAD = NP * T
E_PAD = 163840
ROWS_SC = NP // NS
HH = H // 2
HQ = H // 4
BLK = 1024

F32 = jnp.float32


def _dot_nt(a, b):
    return lax.dot_general(a, b, (((1,), (1,)), ((), ())),
                           preferred_element_type=F32)


def _dot_nn(a, b):
    return lax.dot_general(a, b, (((1,), (0,)), ((), ())),
                           preferred_element_type=F32)


@functools.lru_cache(maxsize=None)
def _mesh():
    return plsc.VectorSubcoreMesh(core_axis_name="c", subcore_axis_name="s",
                                  num_cores=NC, num_subcores=NS)


TCH = TOK_PAD // NW // 128


def _k1_body(tok_idx, wemb, nf_out, idx_v, buf, nfb, semg, semw):
    wid = lax.axis_index("s") * NC + lax.axis_index("c")
    base = wid * (TOK_PAD // NW)
    pltpu.sync_copy(tok_idx.at[pl.ds(base, TOK_PAD // NW)], idx_v)

    def chunk(c, carry):
        pltpu.async_copy(wemb.at[idx_v.at[pl.ds(c * 128, 128)]], buf, semg).wait()

        @pl.when(c > 0)
        def _():
            pltpu.make_async_copy(nfb, nf_out.at[pl.ds(0, 32)], semw).wait()

        def node(n, carry2):
            r = n * 4
            for l in range(H // 16):
                sl = pl.ds(l * 16, 16)
                acc = (buf[r, sl] + buf[r + 1, sl]) + (buf[r + 2, sl] + buf[r + 3, sl])
                nfb[n, sl] = acc * 0.25
            return carry2

        lax.fori_loop(0, 32, node, 0)
        pltpu.async_copy(
            nfb, nf_out.at[pl.ds(wid * 320 + c * 32, 32)], semw)
        return carry

    lax.fori_loop(0, TCH, chunk, 0)
    pltpu.make_async_copy(nfb, nf_out.at[pl.ds(0, 32)], semw).wait()


@functools.lru_cache(maxsize=None)
def _k1_kernel():
    return pl.kernel(
        _k1_body,
        out_type=jax.ShapeDtypeStruct((NP, H), F32),
        mesh=_mesh(),
        scratch_types=[
            pltpu.VMEM((TOK_PAD // NW,), jnp.int32),
            pltpu.VMEM((128, H), F32),
            pltpu.VMEM((32, H), F32),
            pltpu.SemaphoreType.DMA,
            pltpu.SemaphoreType.DMA,
        ],
    )


def _k1_call(tok_idx, wemb):
    return _k1_kernel()(tok_idx, wemb)


def _k2_body(nf_in, wmsg, wh, bh, x1a_o, x1b_o, gh_o):
    nf = nf_in[...]
    x1 = _dot_nn(nf, wmsg[...])
    x1a_o[...] = x1[:, :HH]
    x1b_o[...] = x1[:, HH:]
    gh_o[...] = _dot_nt(nf, wh[...]) + bh[...]


def _k2_call(nf, W_msg, Wh, bh_row):
    grid = (NP // BLK,)
    return pl.pallas_call(
        _k2_body,
        grid=grid,
        in_specs=[
            pl.BlockSpec((BLK, H), lambda i: (i, 0)),
            pl.BlockSpec((H, H), lambda i: (0, 0)),
            pl.BlockSpec((3 * H, H), lambda i: (0, 0)),
            pl.BlockSpec((1, 3 * H), lambda i: (0, 0)),
        ],
        out_specs=[
            pl.BlockSpec((BLK, HH), lambda i: (i, 0)),
            pl.BlockSpec((BLK, HH), lambda i: (i, 0)),
            pl.BlockSpec((BLK, 3 * H), lambda i: (i, 0)),
        ],
        out_shape=[
            jax.ShapeDtypeStruct((NP, HH), F32),
            jax.ShapeDtypeStruct((NP, HH), F32),
            jax.ShapeDtypeStruct((NP, 3 * H), F32),
        ],
    )(nf, W_msg, Wh, bh_row)


NBUF = 1
CH_W = (E_PAD // NS) // 128


SLAB = NP + 128
ROWS_Z = SLAB // NS
ROWS_W = NP // NS


def _k3_body(src2, dst2, x1a, x1b, zrows, agg_a, agg_b,
             idx_s, idx_d, ib0, rows0, semg, sems, shared):
    cid = lax.axis_index("c")
    sid = lax.axis_index("s")

    row_base = sid * CH_W
    pltpu.sync_copy(src2.at[pl.ds(row_base, CH_W)], idx_s)
    pltpu.sync_copy(dst2.at[pl.ds(row_base, CH_W)], idx_d)
    pltpu.sync_copy(zrows, shared.at[pl.ds(sid * ROWS_Z, ROWS_Z)])
    plsc.subcore_barrier()

    def run(table, out_ref):
        def step(j, carry):
            @pl.when(j > 0)
            def _():
                pltpu.make_async_copy(rows0, shared.at[ib0], sems).wait()
            for l in range(8):
                ib0[pl.ds(l * 16, 16)] = idx_d[j, pl.ds(l * 16, 16)]
            pltpu.async_copy(table.at[idx_s.at[j]], rows0, semg).wait()
            pltpu.async_copy(rows0, shared.at[ib0], sems, add=True)
            return carry

        lax.fori_loop(0, CH_W, step, 0)
        pltpu.make_async_copy(rows0, shared.at[ib0], sems).wait()
        plsc.subcore_barrier()
        pltpu.sync_copy(shared.at[pl.ds(sid * ROWS_W, ROWS_W)],
                        out_ref.at[pl.ds(sid * ROWS_W, ROWS_W)])

    @pl.when(cid == 0)
    def _():
        run(x1a, agg_a)

    @pl.when(cid == 1)
    def _():
        run(x1b, agg_b)


@functools.lru_cache(maxsize=None)
def _k3_kernel():
    return pl.kernel(
        _k3_body,
        out_type=[
            jax.ShapeDtypeStruct((NP, HH), F32),
            jax.ShapeDtypeStruct((NP, HH), F32),
        ],
        mesh=_mesh(),
        scratch_types=[
            pltpu.VMEM((CH_W, 128), jnp.int32),
            pltpu.VMEM((CH_W, 128), jnp.int32),
            pltpu.VMEM((128,), jnp.int32),
            pltpu.VMEM((128, HH), F32),
            pltpu.SemaphoreType.DMA,
            pltpu.SemaphoreType.DMA,
            pltpu.VMEM_SHARED((SLAB, HH), F32),
        ],
    )


def _k3_call(src2, dst2, x1a, x1b, zrows):
    return _k3_kernel()(src2, dst2, x1a, x1b, zrows)


def _k4_body(agg_a, agg_b, gh, nf, wi, bi, h_o, sums_o):
    i = pl.program_id(0)
    agg = jnp.concatenate([agg_a[...], agg_b[...]], axis=1)
    gi = _dot_nt(agg, wi[...]) + bi[...]
    ghv = gh[...]
    r = jax.nn.sigmoid(gi[:, 0:H] + ghv[:, 0:H])
    z = jax.nn.sigmoid(gi[:, H:2 * H] + ghv[:, H:2 * H])
    n = jnp.tanh(gi[:, 2 * H:] + r * ghv[:, 2 * H:])
    h = (1.0 - z) * n + z * nf[...]
    h_o[...] = h
    row = lax.broadcasted_iota(jnp.int32, (BLK, 1), 0) + i * BLK
    hm = jnp.where(row < N, h, 0.0)
    s = jnp.sum(hm, axis=0, keepdims=True)
    ss = jnp.sum(hm * hm, axis=0, keepdims=True)
    pack = jnp.concatenate([s, ss, jnp.zeros((6, H), dtype=F32)], axis=0)

    @pl.when(i == 0)
    def _():
        sums_o[...] = pack

    @pl.when(i > 0)
    def _():
        sums_o[...] = sums_o[...] + pack


def _k4_call(agg_a, agg_b, gh, nf, Wi, bi_row):
    grid = (NP // BLK,)
    return pl.pallas_call(
        _k4_body,
        grid=grid,
        in_specs=[
            pl.BlockSpec((BLK, HH), lambda i: (i, 0)),
            pl.BlockSpec((BLK, HH), lambda i: (i, 0)),
            pl.BlockSpec((BLK, 3 * H), lambda i: (i, 0)),
            pl.BlockSpec((BLK, H), lambda i: (i, 0)),
            pl.BlockSpec((3 * H, H), lambda i: (0, 0)),
            pl.BlockSpec((1, 3 * H), lambda i: (0, 0)),
        ],
        out_specs=[
            pl.BlockSpec((BLK, H), lambda i: (i, 0)),
            pl.BlockSpec((8, H), lambda i: (0, 0)),
        ],
        out_shape=[
            jax.ShapeDtypeStruct((NP, H), F32),
            jax.ShapeDtypeStruct((8, H), F32),
        ],
    )(agg_a, agg_b, gh, nf, Wi, bi_row)


def _k5_body(e1_idx, rel_idx, h, rel_emb, he, re, idx_v, rows_v, sem):
    wid = lax.axis_index("s") * NC + lax.axis_index("c")
    per = B // NW
    base = wid * per
    pltpu.sync_copy(e1_idx.at[pl.ds(base, per)], idx_v)
    pltpu.async_copy(h.at[idx_v], rows_v, sem).wait()
    pltpu.sync_copy(rows_v, he.at[pl.ds(base, per)])
    pltpu.sync_copy(rel_idx.at[pl.ds(base, per)], idx_v)
    pltpu.async_copy(rel_emb.at[idx_v], rows_v, sem).wait()
    pltpu.sync_copy(rows_v, re.at[pl.ds(base, per)])


@functools.lru_cache(maxsize=None)
def _k5_kernel():
    return pl.kernel(
        _k5_body,
        out_type=[
            jax.ShapeDtypeStruct((B, H), F32),
            jax.ShapeDtypeStruct((B, H), F32),
        ],
        mesh=_mesh(),
        scratch_types=[
            pltpu.VMEM((B // NW,), jnp.int32),
            pltpu.VMEM((B // NW, H), F32),
            pltpu.SemaphoreType.DMA,
        ],
    )


def _k5_call(e1_idx, rel_idx, h, rel_emb):
    return _k5_kernel()(e1_idx, rel_idx, h, rel_emb)


def _k6_body(he, re, sums, gamma, beta, h, e2, logits_o, loss_o):
    i = pl.program_id(0)
    ng = pl.num_programs(0)
    inv_n = 1.0 / N
    mean = sums[0:1, :] * inv_n
    var = sums[1:2, :] * inv_n - mean * mean
    sc = lax.rsqrt(var + 1e-5) * gamma[...]
    q = ((he[...] - mean) * sc + beta[...]) * re[...]
    hb = (h[...] - mean) * sc + beta[...]
    lg = jax.nn.sigmoid(_dot_nt(q, hb))
    logits_o[...] = lg
    p = jnp.clip(lg, 1e-7, 1.0 - 1e-7)
    e2v = e2[...]
    col = lax.broadcasted_iota(jnp.int32, (1, BLK), 1) + i * BLK
    term = e2v * jnp.log(p) + (1.0 - e2v) * jnp.log(1.0 - p)
    part = jnp.sum(jnp.where(col < N, term, 0.0))

    @pl.when(i == 0)
    def _():
        loss_o[0, 0] = part

    @pl.when(i > 0)
    def _():
        loss_o[0, 0] = loss_o[0, 0] + part

    @pl.when(i == ng - 1)
    def _():
        loss_o[0, 0] = loss_o[0, 0] * (-1.0 / (B * N))


def _k6_call(he, re, sums, gamma_row, beta_row, h, e2_pad):
    grid = (NP // BLK,)
    return pl.pallas_call(
        _k6_body,
        grid=grid,
        in_specs=[
            pl.BlockSpec((B, H), lambda i: (0, 0)),
            pl.BlockSpec((B, H), lambda i: (0, 0)),
            pl.BlockSpec((8, H), lambda i: (0, 0)),
            pl.BlockSpec((1, H), lambda i: (0, 0)),
            pl.BlockSpec((1, H), lambda i: (0, 0)),
            pl.BlockSpec((BLK, H), lambda i: (i, 0)),
            pl.BlockSpec((B, BLK), lambda i: (0, i)),
        ],
        out_specs=[
            pl.BlockSpec((B, BLK), lambda i: (0, i)),
            pl.BlockSpec(memory_space=pltpu.SMEM),
        ],
        out_shape=[
            jax.ShapeDtypeStruct((B, NP), F32),
            jax.ShapeDtypeStruct((1, 1), F32),
        ],
    )(he, re, sums, gamma_row, beta_row, h, e2_pad)


def kernel(node_token_idx, edge_index, e1, rel, e2_multi, word_emb,
           W_msg, Wi, Wh, bi, bh, bn_gamma, bn_beta, rel_emb):
    tok_flat = jnp.concatenate(
        [node_token_idx.reshape(-1),
         jnp.zeros((TOK_PAD - N * T,), jnp.int32)])
    src2 = jnp.concatenate(
        [edge_index[0], jnp.zeros((E_PAD - E,), jnp.int32)]).reshape(-1, 128)
    dst2 = jnp.concatenate(
        [edge_index[1], jnp.full((E_PAD - E,), NP, jnp.int32)]).reshape(-1, 128)

    nf = _k1_call(tok_flat, word_emb)

    x1a, x1b, gh = _k2_call(nf, W_msg, Wh, bh.reshape(1, 3 * H))

    zrows = jnp.zeros((ROWS_Z, HH), F32)
    agg_a, agg_b = _k3_call(src2, dst2, x1a, x1b, zrows)

    h, sums = _k4_call(agg_a, agg_b, gh, nf, Wi, bi.reshape(1, 3 * H))

    he, re = _k5_call(e1[:, 0], rel[:, 0], h, rel_emb)

    logits_pad, loss = _k6_call(he, re, sums, bn_gamma.reshape(1, H),
                                bn_beta.reshape(1, H), h, e2_multi)
    return logits_pad[:, :N], loss[0, 0]


# speedup vs baseline: 2.6550x; 1.0494x over previous
import functools

import jax
import jax.numpy as jnp
from jax import lax
from jax.experimental import pallas as pl
from jax.experimental.pallas import tpu as pltpu
from jax.experimental.pallas import tpu_sc as plsc

N = 10000
E = 160000
H = 256
B = 1024
R = 64
V = 50000
T = 4

NC = 2
NS = 16
NW = NC * NS

NP = 10240
TOK_PAD = NP * T
E_PAD = 163840
ROWS_SC = NP // NS
HH = H // 2
HQ = H // 4
BLK = 1024

F32 = jnp.float32


def _dot_nt(a, b):
    return lax.dot_general(a, b, (((1,), (1,)), ((), ())),
                           preferred_element_type=F32)


def _dot_nn(a, b):
    return lax.dot_general(a, b, (((1,), (0,)), ((), ())),
                           preferred_element_type=F32)


@functools.lru_cache(maxsize=None)
def _mesh():
    return plsc.VectorSubcoreMesh(core_axis_name="c", subcore_axis_name="s",
                                  num_cores=NC, num_subcores=NS)


TCH = TOK_PAD // NW // 128


def _k1_body(tok_idx, wemb, nf_out, idx_v, buf_a, buf_b, nfb_a, nfb_b,
             semg_a, semg_b, semw_a, semw_b):
    wid = lax.axis_index("s") * NC + lax.axis_index("c")
    base = wid * (TOK_PAD // NW)
    nf_base = wid * 320
    pltpu.sync_copy(tok_idx.at[pl.ds(base, TOK_PAD // NW)], idx_v)

    def gather(c, buf, semg):
        pltpu.async_copy(wemb.at[idx_v.at[pl.ds(c * 128, 128)]], buf, semg)

    def compute(buf, nfb):
        def node(n, carry2):
            r = n * 4
            for l in range(H // 16):
                sl = pl.ds(l * 16, 16)
                acc = (buf[r, sl] + buf[r + 1, sl]) + (buf[r + 2, sl] + buf[r + 3, sl])
                nfb[n, sl] = acc * 0.25
            return carry2

        lax.fori_loop(0, 32, node, 0)

    gather(0, buf_a, semg_a)

    def pair(k, carry):
        c = k * 2
        gather(c + 1, buf_b, semg_b)
        pltpu.make_async_copy(wemb.at[idx_v.at[pl.ds(0, 128)]], buf_a, semg_a).wait()

        @pl.when(k > 0)
        def _():
            pltpu.make_async_copy(nfb_a, nf_out.at[pl.ds(0, 32)], semw_a).wait()
        compute(buf_a, nfb_a)
        pltpu.async_copy(nfb_a, nf_out.at[pl.ds(nf_base + c * 32, 32)], semw_a)

        @pl.when(c + 2 < TCH)
        def _():
            gather(c + 2, buf_a, semg_a)
        pltpu.make_async_copy(wemb.at[idx_v.at[pl.ds(0, 128)]], buf_b, semg_b).wait()

        @pl.when(k > 0)
        def _():
            pltpu.make_async_copy(nfb_b, nf_out.at[pl.ds(0, 32)], semw_b).wait()
        compute(buf_b, nfb_b)
        pltpu.async_copy(nfb_b, nf_out.at[pl.ds(nf_base + (c + 1) * 32, 32)], semw_b)
        return carry

    lax.fori_loop(0, TCH // 2, pair, 0)
    pltpu.make_async_copy(nfb_a, nf_out.at[pl.ds(0, 32)], semw_a).wait()
    pltpu.make_async_copy(nfb_b, nf_out.at[pl.ds(0, 32)], semw_b).wait()


@functools.lru_cache(maxsize=None)
def _k1_kernel():
    return pl.kernel(
        _k1_body,
        out_type=jax.ShapeDtypeStruct((NP, H), F32),
        mesh=_mesh(),
        scratch_types=[
            pltpu.VMEM((TOK_PAD // NW,), jnp.int32),
            pltpu.VMEM((128, H), F32),
            pltpu.VMEM((128, H), F32),
            pltpu.VMEM((32, H), F32),
            pltpu.VMEM((32, H), F32),
            pltpu.SemaphoreType.DMA,
            pltpu.SemaphoreType.DMA,
            pltpu.SemaphoreType.DMA,
            pltpu.SemaphoreType.DMA,
        ],
    )


def _k1_call(tok_idx, wemb):
    return _k1_kernel()(tok_idx, wemb)


def _k2_body(nf_in, wmsg, wh, bh, x1a_o, x1b_o, gh_o):
    nf = nf_in[...]
    x1 = _dot_nn(nf, wmsg[...])
    x1a_o[...] = x1[:, :HH]
    x1b_o[...] = x1[:, HH:]
    gh_o[...] = _dot_nt(nf, wh[...]) + bh[...]


def _k2_call(nf, W_msg, Wh, bh_row):
    grid = (NP // BLK,)
    return pl.pallas_call(
        _k2_body,
        grid=grid,
        in_specs=[
            pl.BlockSpec((BLK, H), lambda i: (i, 0)),
            pl.BlockSpec((H, H), lambda i: (0, 0)),
            pl.BlockSpec((3 * H, H), lambda i: (0, 0)),
            pl.BlockSpec((1, 3 * H), lambda i: (0, 0)),
        ],
        out_specs=[
            pl.BlockSpec((BLK, HH), lambda i: (i, 0)),
            pl.BlockSpec((BLK, HH), lambda i: (i, 0)),
            pl.BlockSpec((BLK, 3 * H), lambda i: (i, 0)),
        ],
        out_shape=[
            jax.ShapeDtypeStruct((NP, HH), F32),
            jax.ShapeDtypeStruct((NP, HH), F32),
            jax.ShapeDtypeStruct((NP, 3 * H), F32),
        ],
    )(nf, W_msg, Wh, bh_row)


NBUF = 1
CH_W = (E_PAD // NS) // 128


SLAB = NP + 128
ROWS_Z = SLAB // NS
ROWS_W = NP // NS


def _k3_body(src2, dst2, x1a, x1b, zrows, agg_a, agg_b,
             idx_s, idx_d, ib0, rows0, semg, sems, shared):
    cid = lax.axis_index("c")
    sid = lax.axis_index("s")

    row_base = sid * CH_W
    pltpu.sync_copy(src2.at[pl.ds(row_base, CH_W)], idx_s)
    pltpu.sync_copy(dst2.at[pl.ds(row_base, CH_W)], idx_d)
    pltpu.sync_copy(zrows, shared.at[pl.ds(sid * ROWS_Z, ROWS_Z)])
    plsc.subcore_barrier()

    def run(table, out_ref):
        def step(j, carry):
            @pl.when(j > 0)
            def _():
                pltpu.make_async_copy(rows0, shared.at[ib0], sems).wait()
            for l in range(8):
                ib0[pl.ds(l * 16, 16)] = idx_d[j, pl.ds(l * 16, 16)]
            pltpu.async_copy(table.at[idx_s.at[j]], rows0, semg).wait()
            pltpu.async_copy(rows0, shared.at[ib0], sems, add=True)
            return carry

        lax.fori_loop(0, CH_W, step, 0)
        pltpu.make_async_copy(rows0, shared.at[ib0], sems).wait()
        plsc.subcore_barrier()
        pltpu.sync_copy(shared.at[pl.ds(sid * ROWS_W, ROWS_W)],
                        out_ref.at[pl.ds(sid * ROWS_W, ROWS_W)])

    @pl.when(cid == 0)
    def _():
        run(x1a, agg_a)

    @pl.when(cid == 1)
    def _():
        run(x1b, agg_b)


@functools.lru_cache(maxsize=None)
def _k3_kernel():
    return pl.kernel(
        _k3_body,
        out_type=[
            jax.ShapeDtypeStruct((NP, HH), F32),
            jax.ShapeDtypeStruct((NP, HH), F32),
        ],
        mesh=_mesh(),
        scratch_types=[
            pltpu.VMEM((CH_W, 128), jnp.int32),
            pltpu.VMEM((CH_W, 128), jnp.int32),
            pltpu.VMEM((128,), jnp.int32),
            pltpu.VMEM((128, HH), F32),
            pltpu.SemaphoreType.DMA,
            pltpu.SemaphoreType.DMA,
            pltpu.VMEM_SHARED((SLAB, HH), F32),
        ],
    )


def _k3_call(src2, dst2, x1a, x1b, zrows):
    return _k3_kernel()(src2, dst2, x1a, x1b, zrows)


def _k4_body(agg_a, agg_b, gh, nf, wi, bi, h_o, sums_o):
    i = pl.program_id(0)
    agg = jnp.concatenate([agg_a[...], agg_b[...]], axis=1)
    gi = _dot_nt(agg, wi[...]) + bi[...]
    ghv = gh[...]
    r = jax.nn.sigmoid(gi[:, 0:H] + ghv[:, 0:H])
    z = jax.nn.sigmoid(gi[:, H:2 * H] + ghv[:, H:2 * H])
    n = jnp.tanh(gi[:, 2 * H:] + r * ghv[:, 2 * H:])
    h = (1.0 - z) * n + z * nf[...]
    h_o[...] = h
    row = lax.broadcasted_iota(jnp.int32, (BLK, 1), 0) + i * BLK
    hm = jnp.where(row < N, h, 0.0)
    s = jnp.sum(hm, axis=0, keepdims=True)
    ss = jnp.sum(hm * hm, axis=0, keepdims=True)
    pack = jnp.concatenate([s, ss, jnp.zeros((6, H), dtype=F32)], axis=0)

    @pl.when(i == 0)
    def _():
        sums_o[...] = pack

    @pl.when(i > 0)
    def _():
        sums_o[...] = sums_o[...] + pack


def _k4_call(agg_a, agg_b, gh, nf, Wi, bi_row):
    grid = (NP // BLK,)
    return pl.pallas_call(
        _k4_body,
        grid=grid,
        in_specs=[
            pl.BlockSpec((BLK, HH), lambda i: (i, 0)),
            pl.BlockSpec((BLK, HH), lambda i: (i, 0)),
            pl.BlockSpec((BLK, 3 * H), lambda i: (i, 0)),
            pl.BlockSpec((BLK, H), lambda i: (i, 0)),
            pl.BlockSpec((3 * H, H), lambda i: (0, 0)),
            pl.BlockSpec((1, 3 * H), lambda i: (0, 0)),
        ],
        out_specs=[
            pl.BlockSpec((BLK, H), lambda i: (i, 0)),
            pl.BlockSpec((8, H), lambda i: (0, 0)),
        ],
        out_shape=[
            jax.ShapeDtypeStruct((NP, H), F32),
            jax.ShapeDtypeStruct((8, H), F32),
        ],
    )(agg_a, agg_b, gh, nf, Wi, bi_row)


def _k5_body(e1_idx, rel_idx, h, rel_emb, he, re, idx_v, rows_v, sem):
    wid = lax.axis_index("s") * NC + lax.axis_index("c")
    per = B // NW
    base = wid * per
    pltpu.sync_copy(e1_idx.at[pl.ds(base, per)], idx_v)
    pltpu.async_copy(h.at[idx_v], rows_v, sem).wait()
    pltpu.sync_copy(rows_v, he.at[pl.ds(base, per)])
    pltpu.sync_copy(rel_idx.at[pl.ds(base, per)], idx_v)
    pltpu.async_copy(rel_emb.at[idx_v], rows_v, sem).wait()
    pltpu.sync_copy(rows_v, re.at[pl.ds(base, per)])


@functools.lru_cache(maxsize=None)
def _k5_kernel():
    return pl.kernel(
        _k5_body,
        out_type=[
            jax.ShapeDtypeStruct((B, H), F32),
            jax.ShapeDtypeStruct((B, H), F32),
        ],
        mesh=_mesh(),
        scratch_types=[
            pltpu.VMEM((B // NW,), jnp.int32),
            pltpu.VMEM((B // NW, H), F32),
            pltpu.SemaphoreType.DMA,
        ],
    )


def _k5_call(e1_idx, rel_idx, h, rel_emb):
    return _k5_kernel()(e1_idx, rel_idx, h, rel_emb)


def _k6_body(he, re, sums, gamma, beta, h, e2, logits_o, loss_o):
    i = pl.program_id(0)
    ng = pl.num_programs(0)
    inv_n = 1.0 / N
    mean = sums[0:1, :] * inv_n
    var = sums[1:2, :] * inv_n - mean * mean
    sc = lax.rsqrt(var + 1e-5) * gamma[...]
    q = ((he[...] - mean) * sc + beta[...]) * re[...]
    hb = (h[...] - mean) * sc + beta[...]
    lg = jax.nn.sigmoid(_dot_nt(q, hb))
    logits_o[...] = lg
    p = jnp.clip(lg, 1e-7, 1.0 - 1e-7)
    e2v = e2[...]
    col = lax.broadcasted_iota(jnp.int32, (1, BLK), 1) + i * BLK
    term = e2v * jnp.log(p) + (1.0 - e2v) * jnp.log(1.0 - p)
    part = jnp.sum(jnp.where(col < N, term, 0.0))

    @pl.when(i == 0)
    def _():
        loss_o[0, 0] = part

    @pl.when(i > 0)
    def _():
        loss_o[0, 0] = loss_o[0, 0] + part

    @pl.when(i == ng - 1)
    def _():
        loss_o[0, 0] = loss_o[0, 0] * (-1.0 / (B * N))


def _k6_call(he, re, sums, gamma_row, beta_row, h, e2_pad):
    grid = (NP // BLK,)
    return pl.pallas_call(
        _k6_body,
        grid=grid,
        in_specs=[
            pl.BlockSpec((B, H), lambda i: (0, 0)),
            pl.BlockSpec((B, H), lambda i: (0, 0)),
            pl.BlockSpec((8, H), lambda i: (0, 0)),
            pl.BlockSpec((1, H), lambda i: (0, 0)),
            pl.BlockSpec((1, H), lambda i: (0, 0)),
            pl.BlockSpec((BLK, H), lambda i: (i, 0)),
            pl.BlockSpec((B, BLK), lambda i: (0, i)),
        ],
        out_specs=[
            pl.BlockSpec((B, BLK), lambda i: (0, i)),
            pl.BlockSpec(memory_space=pltpu.SMEM),
        ],
        out_shape=[
            jax.ShapeDtypeStruct((B, NP), F32),
            jax.ShapeDtypeStruct((1, 1), F32),
        ],
    )(he, re, sums, gamma_row, beta_row, h, e2_pad)


def kernel(node_token_idx, edge_index, e1, rel, e2_multi, word_emb,
           W_msg, Wi, Wh, bi, bh, bn_gamma, bn_beta, rel_emb):
    tok_flat = jnp.concatenate(
        [node_token_idx.reshape(-1),
         jnp.zeros((TOK_PAD - N * T,), jnp.int32)])
    src2 = jnp.concatenate(
        [edge_index[0], jnp.zeros((E_PAD - E,), jnp.int32)]).reshape(-1, 128)
    dst2 = jnp.concatenate(
        [edge_index[1], jnp.full((E_PAD - E,), NP, jnp.int32)]).reshape(-1, 128)

    nf = _k1_call(tok_flat, word_emb)

    x1a, x1b, gh = _k2_call(nf, W_msg, Wh, bh.reshape(1, 3 * H))

    zrows = jnp.zeros((ROWS_Z, HH), F32)
    agg_a, agg_b = _k3_call(src2, dst2, x1a, x1b, zrows)

    h, sums = _k4_call(agg_a, agg_b, gh, nf, Wi, bi.reshape(1, 3 * H))

    he, re = _k5_call(e1[:, 0], rel[:, 0], h, rel_emb)

    logits_pad, loss = _k6_call(he, re, sums, bn_gamma.reshape(1, H),
                                bn_beta.reshape(1, H), h, e2_multi)
    return logits_pad[:, :N], loss[0, 0]
